# baseline (speedup 1.0000x reference)
import functools
import math

import jax
import jax.numpy as jnp
from jax import lax
from jax.experimental import pallas as pl
from jax.experimental.pallas import tpu as pltpu
from jax.experimental.pallas import tpu_sc as plsc

K = 16
KP = 15
N0, N1, N2 = 50000, 12500, 3125
N0P, N1P, N2P = 50176, 12544, 3200
NW = 32


def _lrelu(x):
    return jnp.where(x >= 0, x, 0.1 * x)


@functools.cache
def _tgather_fn(ns, ct, cu, kk, np_, r):
    nchunks = np_ // r
    nt = -(-nchunks // NW)
    jblocks = r // 16

    mesh = plsc.VectorSubcoreMesh(core_axis_name="c", subcore_axis_name="s")

    @functools.partial(
        pl.kernel,
        out_type=jax.ShapeDtypeStruct((kk, cu, np_), jnp.float32),
        mesh=mesh,
        scratch_types=[
            pltpu.VMEM((r * kk,), jnp.int32),
            pltpu.VMEM((r * kk,), jnp.int32),
            pltpu.VMEM((r * kk, ct), jnp.float32),
            pltpu.VMEM((r * kk, ct), jnp.float32),
            pltpu.VMEM((kk, cu, r), jnp.float32),
            pltpu.VMEM((kk, cu, r), jnp.float32),
        ] + [pltpu.SemaphoreType.DMA] * 6,
        compiler_params=pltpu.CompilerParams(
            use_tc_tiling_on_sc=False, needs_layout_passes=False),
    )
    def tg(table_hbm, idx_hbm, out_hbm, i0, i1, r0, r1, o0, o1,
           si0, si1, sg0, sg1, so0, so1):
        wid = lax.axis_index("s") * 2 + lax.axis_index("c")
        lane = lax.iota(jnp.int32, 16)
        idx_v, rows_v, obuf = (i0, i1), (r0, r1), (o0, o1)
        semi, semg, semo = (si0, si1), (sg0, sg1), (so0, so1)

        def cid(t):
            return jnp.minimum(wid * nt + t, nchunks - 1)

        def issue_idx(t, s):
            pltpu.async_copy(
                idx_hbm.at[pl.ds(cid(t) * (r * kk), r * kk)],
                idx_v[s], semi[s])

        def wait_idx(s):
            pltpu.make_async_copy(
                idx_hbm.at[pl.ds(0, r * kk)], idx_v[s], semi[s]).wait()

        def issue_gather(s):
            pltpu.async_copy(table_hbm.at[idx_v[s]], rows_v[s], semg[s])

        def wait_gather(s):
            pltpu.make_async_copy(
                table_hbm.at[idx_v[s]], rows_v[s], semg[s]).wait()

        def issue_out(t, s):
            pltpu.async_copy(
                obuf[s], out_hbm.at[:, :, pl.ds(cid(t) * r, r)], semo[s])

        def wait_out(s):
            pltpu.make_async_copy(
                obuf[s], out_hbm.at[:, :, pl.ds(0, r)], semo[s]).wait()

        def transpose(s):
            rv, ob = rows_v[s], obuf[s]

            def c_body(c, _):
                cvec = jnp.full((16,), 0, jnp.int32) + c

                def j_body(jb, _):
                    rbase = lane * kk + jb * (16 * kk)
                    for k in range(kk):
                        v = plsc.load_gather(rv, [rbase + k, cvec])
                        ob[k, c, pl.ds(jb * 16, 16)] = v
                    return 0

                lax.fori_loop(0, jblocks, j_body, 0, unroll=False)
                return 0

            lax.fori_loop(0, cu, c_body, 0, unroll=False)

        issue_idx(0, 0)
        wait_idx(0)
        issue_gather(0)
        if nt > 1:
            issue_idx(1, 1)
        for t in range(nt):
            s = t % 2
            s1 = 1 - s
            wait_gather(s)
            if t + 2 < nt:
                issue_idx(t + 2, s)
            if t + 1 < nt:
                wait_idx(s1)
                issue_gather(s1)
            if t >= 2:
                wait_out(s)
            transpose(s)
            issue_out(t, s)
        wait_out((nt - 1) % 2)
        if nt >= 2:
            wait_out((nt - 2) % 2)

    return tg


def _tgather(table, idx_flat, cu, kk):
    ns, ct = table.shape
    np_ = idx_flat.shape[0] // kk
    budget = 460 * 1024
    r = 16
    for cand in (512, 256, 128, 64, 32, 16):
        if (8 * kk * (ct + cu + 1)) * cand <= budget and np_ % cand == 0:
            r = cand
            break
    return _tgather_fn(ns, ct, cu, kk, np_, r)(table, idx_flat)


@functools.cache
def _rgather_fn(v, d, bp, rb):
    nchunks = bp // rb
    nt = -(-nchunks // NW)
    mesh = plsc.VectorSubcoreMesh(core_axis_name="c", subcore_axis_name="s")

    @functools.partial(
        pl.kernel,
        out_type=jax.ShapeDtypeStruct((bp, d), jnp.float32),
        mesh=mesh,
        scratch_types=[
            pltpu.VMEM((rb,), jnp.int32),
            pltpu.VMEM((rb,), jnp.int32),
            pltpu.VMEM((rb, d), jnp.float32),
            pltpu.VMEM((rb, d), jnp.float32),
        ] + [pltpu.SemaphoreType.DMA] * 6,
        compiler_params=pltpu.CompilerParams(
            use_tc_tiling_on_sc=False, needs_layout_passes=False),
    )
    def rg(table_hbm, idx_hbm, out_hbm, i0, i1, r0, r1,
           si0, si1, sg0, sg1, so0, so1):
        wid = lax.axis_index("s") * 2 + lax.axis_index("c")
        idx_v, rows_v = (i0, i1), (r0, r1)
        semi, semg, semo = (si0, si1), (sg0, sg1), (so0, so1)

        def cid(t):
            return jnp.minimum(wid * nt + t, nchunks - 1)

        def issue_idx(t, s):
            pltpu.async_copy(idx_hbm.at[pl.ds(cid(t) * rb, rb)],
                             idx_v[s], semi[s])

        def wait_idx(s):
            pltpu.make_async_copy(idx_hbm.at[pl.ds(0, rb)],
                                  idx_v[s], semi[s]).wait()

        def issue_gather(s):
            pltpu.async_copy(table_hbm.at[idx_v[s]], rows_v[s], semg[s])

        def wait_gather(s):
            pltpu.make_async_copy(table_hbm.at[idx_v[s]], rows_v[s],
                                  semg[s]).wait()

        def issue_out(t, s):
            pltpu.async_copy(rows_v[s], out_hbm.at[pl.ds(cid(t) * rb, rb)],
                             semo[s])

        def wait_out(s):
            pltpu.make_async_copy(rows_v[s], out_hbm.at[pl.ds(0, rb)],
                                  semo[s]).wait()

        issue_idx(0, 0)
        wait_idx(0)
        issue_gather(0)
        if nt > 1:
            issue_idx(1, 1)
        for t in range(nt):
            s = t % 2
            s1 = 1 - s
            wait_gather(s)
            if t + 2 < nt:
                issue_idx(t + 2, s)
            if t + 1 < nt:
                wait_idx(s1)
                if t >= 1:
                    wait_out(s1)
                issue_gather(s1)
            issue_out(t, s)
        wait_out((nt - 1) % 2)
        if nt >= 2:
            wait_out((nt - 2) % 2)

    return rg


def _rgather(table, idx):
    v, d = table.shape
    bp = idx.shape[0]
    rb = 128 if d > 128 else 256
    while bp % rb:
        rb //= 2
    return _rgather_fn(v, d, bp, rb)(table, idx)


@functools.cache
def _mpgather_fn(ns, ct, np_, r):
    nchunks = np_ // r
    nt = -(-nchunks // NW)
    cblocks = ct // 16
    mesh = plsc.VectorSubcoreMesh(core_axis_name="c", subcore_axis_name="s")

    @functools.partial(
        pl.kernel,
        out_type=jax.ShapeDtypeStruct((np_, ct), jnp.float32),
        mesh=mesh,
        scratch_types=[
            pltpu.VMEM((r * K,), jnp.int32),
            pltpu.VMEM((r * K,), jnp.int32),
            pltpu.VMEM((r * K, ct), jnp.float32),
            pltpu.VMEM((r * K, ct), jnp.float32),
            pltpu.VMEM((r, ct), jnp.float32),
            pltpu.VMEM((r, ct), jnp.float32),
        ] + [pltpu.SemaphoreType.DMA] * 6,
        compiler_params=pltpu.CompilerParams(
            use_tc_tiling_on_sc=False, needs_layout_passes=False),
    )
    def mp(table_hbm, idx_hbm, out_hbm, i0, i1, r0, r1, o0, o1,
           si0, si1, sg0, sg1, so0, so1):
        wid = lax.axis_index("s") * 2 + lax.axis_index("c")
        idx_v, rows_v, obuf = (i0, i1), (r0, r1), (o0, o1)
        semi, semg, semo = (si0, si1), (sg0, sg1), (so0, so1)

        def cid(t):
            return jnp.minimum(wid * nt + t, nchunks - 1)

        def issue_idx(t, s):
            pltpu.async_copy(idx_hbm.at[pl.ds(cid(t) * (r * K), r * K)],
                             idx_v[s], semi[s])

        def wait_idx(s):
            pltpu.make_async_copy(idx_hbm.at[pl.ds(0, r * K)],
                                  idx_v[s], semi[s]).wait()

        def issue_gather(s):
            pltpu.async_copy(table_hbm.at[idx_v[s]], rows_v[s], semg[s])

        def wait_gather(s):
            pltpu.make_async_copy(table_hbm.at[idx_v[s]], rows_v[s],
                                  semg[s]).wait()

        def issue_out(t, s):
            pltpu.async_copy(obuf[s], out_hbm.at[pl.ds(cid(t) * r, r)],
                             semo[s])

        def wait_out(s):
            pltpu.make_async_copy(obuf[s], out_hbm.at[pl.ds(0, r)],
                                  semo[s]).wait()

        def pool(s):
            rv, ob = rows_v[s], obuf[s]

            def j_body(j, _):
                def c_body(cb, _):
                    m = rv[j * K, pl.ds(cb * 16, 16)]
                    for k in range(1, K):
                        m = jnp.maximum(m, rv[j * K + k, pl.ds(cb * 16, 16)])
                    ob[j, pl.ds(cb * 16, 16)] = m
                    return 0

                lax.fori_loop(0, cblocks, c_body, 0, unroll=False)
                return 0

            lax.fori_loop(0, r, j_body, 0, unroll=False)

        issue_idx(0, 0)
        wait_idx(0)
        issue_gather(0)
        if nt > 1:
            issue_idx(1, 1)
        for t in range(nt):
            s = t % 2
            s1 = 1 - s
            wait_gather(s)
            if t + 2 < nt:
                issue_idx(t + 2, s)
            if t + 1 < nt:
                wait_idx(s1)
                issue_gather(s1)
            if t >= 2:
                wait_out(s)
            pool(s)
            issue_out(t, s)
        wait_out((nt - 1) % 2)
        if nt >= 2:
            wait_out((nt - 2) % 2)

    return mp


def _mpgather(table, idx_flat):
    ns, ct = table.shape
    np_ = idx_flat.shape[0] // K
    budget = 460 * 1024
    r = 16
    for cand in (128, 64, 32, 16):
        if (8 * K * (ct + 1) + 8 * ct) * cand <= budget and np_ % cand == 0:
            r = cand
            break
    return _mpgather_fn(ns, ct, np_, r)(table, idx_flat)


def _kpconv_call(mode, gt, qt, kptt, wflat, bc, sigma, c, o,
                 extras, np_, bn, cout):
    cu = gt.shape[1]
    if mode == "b0":
        poff, feat_off = 0, 3
    else:
        poff = c

    def body(*refs):
        if mode == "b0":
            gt_ref, qt_ref, kptt_ref, wflat_ref, bc_ref, o_ref = refs
        elif mode == "ws":
            (gt_ref, qt_ref, kptt_ref, wflat_ref, bc_ref,
             x_ref, w2_ref, b2_ref, ws_ref, bs_ref, o_ref) = refs
        else:
            (gt_ref, qt_ref, kptt_ref, wflat_ref, bc_ref,
             w2_ref, b2_ref, sc_ref, o_ref) = refs

        gt_b = gt_ref[...]
        qt_b = qt_ref[...]
        kptt_b = kptt_ref[...]
        pt = gt_b[:, poff:poff + 3, :]
        rel = pt - qt_b[None, :, :]
        sq = None
        for d in range(3):
            reld = jnp.broadcast_to(rel[:, d:d + 1, :],
                                    (K, KP, bn)).reshape(K * KP, bn)
            diff = reld - kptt_b[:, d:d + 1]
            sq = diff * diff if sq is None else sq + diff * diff
        infl = jnp.maximum(0.0, 1.0 - jnp.sqrt(sq + 1e-12) / sigma)

        if mode == "b0":
            yg = gt_b[:, feat_off:feat_off + 1, :]
        else:
            yg = gt_b[:, 0:c, :]
        rows = []
        for p in range(KP):
            acc = infl[p, :][None, :] * yg[0]
            for k in range(1, K):
                acc = acc + infl[k * KP + p, :][None, :] * yg[k]
            rows.append(acc)
        wt = jnp.concatenate(rows, axis=0)
        y = lax.dot_general(wt, wflat_ref[...],
                            (((0,), (0,)), ((), ())),
                            preferred_element_type=jnp.float32,
                    precision=lax.Precision.HIGHEST)
        y = _lrelu(y + bc_ref[...])
        if mode == "b0":
            o_ref[...] = y
            return
        y = jnp.dot(y, w2_ref[...],
                    preferred_element_type=jnp.float32,
                    precision=lax.Precision.HIGHEST) + b2_ref[...]
        if mode == "ws":
            scp = jnp.dot(x_ref[...], ws_ref[...],
                          preferred_element_type=jnp.float32,
                    precision=lax.Precision.HIGHEST) + bs_ref[...]
        else:
            scp = sc_ref[...]
        o_ref[...] = _lrelu(y + scp)

    cc = 1 if mode == "b0" else c
    in_specs = [
        pl.BlockSpec((K, cu, bn), lambda i: (0, 0, i)),
        pl.BlockSpec((3, bn), lambda i: (0, i)),
        pl.BlockSpec((K * KP, 3), lambda i: (0, 0)),
        pl.BlockSpec((KP * cc, o), lambda i: (0, 0)),
        pl.BlockSpec((1, o), lambda i: (0, 0)),
    ]
    args = [gt, qt, kptt, wflat, bc]
    if mode == "ws":
        x, w2, b2, ws, bs = extras
        cin = x.shape[1]
        in_specs += [
            pl.BlockSpec((bn, cin), lambda i: (i, 0)),
            pl.BlockSpec((o, cout), lambda i: (0, 0)),
            pl.BlockSpec((1, cout), lambda i: (0, 0)),
            pl.BlockSpec((cin, cout), lambda i: (0, 0)),
            pl.BlockSpec((1, cout), lambda i: (0, 0)),
        ]
        args += [x, w2, b2.reshape(1, -1), ws, bs.reshape(1, -1)]
    elif mode == "pool":
        w2, b2, scrow = extras
        in_specs += [
            pl.BlockSpec((o, cout), lambda i: (0, 0)),
            pl.BlockSpec((1, cout), lambda i: (0, 0)),
            pl.BlockSpec((bn, cout), lambda i: (i, 0)),
        ]
        args += [w2, b2.reshape(1, -1), scrow]

    return pl.pallas_call(
        body,
        grid=(np_ // bn,),
        in_specs=in_specs,
        out_specs=pl.BlockSpec((bn, cout), lambda i: (i, 0)),
        out_shape=jax.ShapeDtypeStruct((np_, cout), jnp.float32),
    )(*args)


def _unary_call(xs, ws, b, act, bn):
    np_ = xs[0].shape[0]
    o = ws[0].shape[1]

    def body(*refs):
        n_in = len(xs)
        acc = refs[2 * n_in][...]
        for i in range(n_in):
            acc = acc + jnp.dot(refs[i][...], refs[n_in + i][...],
                                preferred_element_type=jnp.float32,
                    precision=lax.Precision.HIGHEST)
        refs[-1][...] = _lrelu(acc) if act else acc

    in_specs = [pl.BlockSpec((bn, x.shape[1]), lambda i: (i, 0)) for x in xs]
    in_specs += [pl.BlockSpec(w.shape, lambda i: (0, 0)) for w in ws]
    in_specs += [pl.BlockSpec((1, o), lambda i: (0, 0))]
    return pl.pallas_call(
        body,
        grid=(np_ // bn,),
        in_specs=in_specs,
        out_specs=pl.BlockSpec((bn, o), lambda i: (i, 0)),
        out_shape=jax.ShapeDtypeStruct((np_, o), jnp.float32),
    )(*xs, *ws, b.reshape(1, -1))


def _pad_rows(a, n):
    return jnp.pad(a, ((0, n - a.shape[0]),) + ((0, 0),) * (a.ndim - 1))


def _mktable(parts, ctot):
    t = jnp.concatenate(parts, axis=1)
    return jnp.pad(t, ((0, 0), (0, ctot - t.shape[1])))


def kernel(features, points0, points1, points2, params, neighbors0,
           neighbors1, neighbors2, pools0, pools1, upsamples0, upsamples1):
    p = params
    kp0 = p['kp']

    nb0 = _pad_rows(neighbors0, N0P).reshape(-1)
    nb1 = _pad_rows(neighbors1, N1P).reshape(-1)
    nb2 = _pad_rows(neighbors2, N2P).reshape(-1)
    pl0 = _pad_rows(pools0, N1P).reshape(-1)
    pl1 = _pad_rows(pools1, N2P).reshape(-1)
    up0 = _pad_rows(upsamples0[:, 0], N0P)
    up1 = _pad_rows(upsamples1[:, 0], N1P)
    ia0 = jnp.arange(N0P, dtype=jnp.int32)
    ia1 = jnp.arange(N1P, dtype=jnp.int32)
    ia2 = jnp.arange(N2P, dtype=jnp.int32)

    pts0 = _pad_rows(points0, N0P)
    pts1 = _pad_rows(points1, N1P)
    pts2 = _pad_rows(points2, N2P)
    feat = _pad_rows(features, N0P)

    consts = []
    for lvl in range(3):
        kpts = kp0 * (2.0 ** lvl)
        sig = 0.3 * (2.0 ** lvl)
        kptt = jnp.tile(kpts, (K, 1))
        consts.append((kptt, sig))

    def wflat(wc):
        return wc.reshape(KP * wc.shape[1], wc.shape[2])

    bn0, bn1, bn2 = 896, 896, 640

    t0 = _mktable([pts0, feat], 16)
    qt0 = _tgather(t0, ia0, 3, 1).reshape(3, N0P)
    gt_b0 = _tgather(t0, nb0, 4, K)
    kptt0, s0 = consts[0]
    x0 = _kpconv_call("b0", gt_b0, qt0, kptt0,
                      p['b0']['w'].reshape(KP, 32), p['b0']['b'].reshape(1, -1),
                      s0, 1, 32, None, N0P, bn0, 32)

    y1 = _unary_call([x0], [p['b1']['w1']], p['b1']['b1'], True, bn0)
    gt1 = _tgather(_mktable([y1, pts0], 32), nb0, 19, K)
    x1 = _kpconv_call("ws", gt1, qt0, kptt0, wflat(p['b1']['wc']),
                      p['b1']['bc'].reshape(1, -1), s0, 16, 16,
                      (x0, p['b1']['w2'], p['b1']['b2'], p['b1']['ws'],
                       p['b1']['bs']), N0P, bn0, 64)

    y2 = _unary_call([x1], [p['b2']['w1']], p['b2']['b1'], True, bn0)
    qt1 = _tgather(_mktable([pts1], 16), ia1, 3, 1).reshape(3, N1P)
    gt2 = _tgather(_mktable([y2, pts0], 32), pl0, 19, K)
    sc2 = _mpgather(x1, pl0)
    x2 = _kpconv_call("pool", gt2, qt1, kptt0, wflat(p['b2']['wc']),
                      p['b2']['bc'].reshape(1, -1), s0, 16, 16,
                      (p['b2']['w2'], p['b2']['b2'], sc2), N1P, bn1, 64)

    y3 = _unary_call([x2], [p['b3']['w1']], p['b3']['b1'], True, bn1)
    gt3 = _tgather(_mktable([y3, pts1], 48), nb1, 35, K)
    kptt1, s1 = consts[1]
    x3 = _kpconv_call("ws", gt3, qt1, kptt1, wflat(p['b3']['wc']),
                      p['b3']['bc'].reshape(1, -1), s1, 32, 32,
                      (x2, p['b3']['w2'], p['b3']['b2'], p['b3']['ws'],
                       p['b3']['bs']), N1P, bn1, 128)

    y4 = _unary_call([x3], [p['b4']['w1']], p['b4']['b1'], True, bn1)
    qt2 = _tgather(_mktable([pts2], 16), ia2, 3, 1).reshape(3, N2P)
    gt4 = _tgather(_mktable([y4, pts1], 48), pl1, 35, K)
    sc4 = _mpgather(x3, pl1)
    x4 = _kpconv_call("pool", gt4, qt2, kptt1, wflat(p['b4']['wc']),
                      p['b4']['bc'].reshape(1, -1), s1, 32, 32,
                      (p['b4']['w2'], p['b4']['b2'], sc4), N2P, bn2, 128)

    y5 = _unary_call([x4], [p['b5']['w1']], p['b5']['b1'], True, bn2)
    gt5 = _tgather(_mktable([y5, pts2], 80), nb2, 67, K)
    kptt2, s2 = consts[2]
    x5 = _kpconv_call("ws", gt5, qt2, kptt2, wflat(p['b5']['wc']),
                      p['b5']['bc'].reshape(1, -1), s2, 64, 64,
                      (x4, p['b5']['w2'], p['b5']['b2'], p['b5']['ws'],
                       p['b5']['bs']), N2P, bn2, 256)

    u1 = _rgather(x5, up1)
    d1 = _unary_call([u1, x3], [p['d1']['w'][:256], p['d1']['w'][256:]],
                     p['d1']['b'], True, bn1)
    u0 = _rgather(d1, up0)
    out = _unary_call([u0, x1], [p['d3']['w'][:128], p['d3']['w'][128:]],
                      p['d3']['b'], False, bn0)
    return out[:N0]


# speedup vs baseline: 1.0255x; 1.0255x over previous
import functools

import jax
import jax.numpy as jnp
from jax import lax
from jax.experimental import pallas as pl
from jax.experimental.pallas import tpu as pltpu
from jax.experimental.pallas import tpu_sc as plsc

K = 16
KP = 15
N0, N1, N2 = 50000, 12500, 3125
N0P, N1P, N2P = 50176, 12544, 3200
NW = 32
_SC_PARAMS = pltpu.CompilerParams(
    use_tc_tiling_on_sc=False, needs_layout_passes=False)
_SC_BUDGET = 480 * 1024


def _lrelu(x):
    return jnp.where(x >= 0, x, 0.1 * x)


def _run_pipeline(nt, nchunks, depth, wid, fetch, wait_gather, wait_out,
                  work, issue_out):
    def cid(t):
        return jnp.minimum(wid * nt + jnp.minimum(t, nt - 1), nchunks - 1)

    for s in range(depth):
        fetch(cid(s), s)
    ntd = -(-nt // depth)

    def body(tt, _):
        for s in range(depth):
            t = tt * depth + s
            wait_gather(s)

            @pl.when(tt > 0)
            def _():
                wait_out(s)

            work(s)
            issue_out(cid(t), s)
            fetch(cid(t + depth), s)
        return 0

    lax.fori_loop(0, ntd, body, 0, unroll=False)
    for s in range(depth):
        wait_gather(s)
        wait_out(s)


@functools.cache
def _tgather_fn(ns, ct, cu, kk, np_, r, depth):
    nchunks = np_ // r
    nt = -(-nchunks // NW)
    jblocks = r // 16

    mesh = plsc.VectorSubcoreMesh(core_axis_name="c", subcore_axis_name="s")

    @functools.partial(
        pl.kernel,
        out_type=jax.ShapeDtypeStruct((kk, cu, np_), jnp.float32),
        mesh=mesh,
        scratch_types=[
            [pltpu.VMEM((r * kk,), jnp.int32)] * depth,
            [pltpu.VMEM((r * kk, ct), jnp.float32)] * depth,
            [pltpu.VMEM((kk, cu, r), jnp.float32)] * depth,
            [pltpu.SemaphoreType.DMA] * depth,
            [pltpu.SemaphoreType.DMA] * depth,
        ],
        compiler_params=_SC_PARAMS,
    )
    def tg(table_hbm, idx_hbm, out_hbm, idx_v, rows_v, obuf, semg, semo):
        wid = lax.axis_index("s") * 2 + lax.axis_index("c")
        lane = lax.iota(jnp.int32, 16)

        def fetch(c, s):
            pltpu.sync_copy(idx_hbm.at[pl.ds(c * (r * kk), r * kk)], idx_v[s])
            pltpu.async_copy(table_hbm.at[idx_v[s]], rows_v[s], semg[s])

        def wait_gather(s):
            pltpu.make_async_copy(
                table_hbm.at[idx_v[s]], rows_v[s], semg[s]).wait()

        def issue_out(c, s):
            pltpu.async_copy(
                obuf[s], out_hbm.at[:, :, pl.ds(c * r, r)], semo[s])

        def wait_out(s):
            pltpu.make_async_copy(
                obuf[s], out_hbm.at[:, :, pl.ds(0, r)], semo[s]).wait()

        def work(s):
            rv, ob = rows_v[s], obuf[s]

            def c_body(c, _):
                cvec = jnp.full((16,), 0, jnp.int32) + c

                def j_body(jb, _):
                    rbase = lane * kk + jb * (16 * kk)
                    for k in range(kk):
                        v = plsc.load_gather(rv, [rbase + k, cvec])
                        ob[k, c, pl.ds(jb * 16, 16)] = v
                    return 0

                lax.fori_loop(0, jblocks, j_body, 0, unroll=False)
                return 0

            lax.fori_loop(0, cu, c_body, 0, unroll=False)

        _run_pipeline(nt, nchunks, depth, wid, fetch, wait_gather, wait_out,
                      work, issue_out)

    return tg


def _tgather(table, idx_flat, cu, kk):
    ns, ct = table.shape
    np_ = idx_flat.shape[0] // kk
    pick = None
    for depth in (4, 2):
        for r in (512, 256, 128, 64, 32, 16):
            if depth * 4 * (r * kk * ct + kk * cu * r + r * kk) <= _SC_BUDGET \
                    and np_ % r == 0:
                if pick is None or r >= 32:
                    pick = (r, depth)
                break
        if pick and pick[0] >= 32:
            break
    r, depth = pick
    return _tgather_fn(ns, ct, cu, kk, np_, r, depth)(table, idx_flat)


@functools.cache
def _rgather_fn(v, d, bp, rb, depth):
    nchunks = bp // rb
    nt = -(-nchunks // NW)
    mesh = plsc.VectorSubcoreMesh(core_axis_name="c", subcore_axis_name="s")

    @functools.partial(
        pl.kernel,
        out_type=jax.ShapeDtypeStruct((bp, d), jnp.float32),
        mesh=mesh,
        scratch_types=[
            [pltpu.VMEM((rb,), jnp.int32)] * depth,
            [pltpu.VMEM((rb, d), jnp.float32)] * depth,
            [pltpu.SemaphoreType.DMA] * depth,
            [pltpu.SemaphoreType.DMA] * depth,
        ],
        compiler_params=_SC_PARAMS,
    )
    def rg(table_hbm, idx_hbm, out_hbm, idx_v, rows_v, semg, semo):
        wid = lax.axis_index("s") * 2 + lax.axis_index("c")

        def cid(t):
            return jnp.minimum(wid * nt + jnp.minimum(t, nt - 1), nchunks - 1)

        def fetch(c, s):
            pltpu.sync_copy(idx_hbm.at[pl.ds(c * rb, rb)], idx_v[s])
            pltpu.async_copy(table_hbm.at[idx_v[s]], rows_v[s], semg[s])

        def wait_gather(s):
            pltpu.make_async_copy(
                table_hbm.at[idx_v[s]], rows_v[s], semg[s]).wait()

        def wait_out(s):
            pltpu.make_async_copy(
                rows_v[s], out_hbm.at[pl.ds(0, rb)], semo[s]).wait()

        for s in range(depth):
            fetch(cid(s), s)
        ntd = -(-nt // depth)

        def body(tt, _):
            for s in range(depth):
                t = tt * depth + s
                wait_gather(s)
                pltpu.async_copy(rows_v[s],
                                 out_hbm.at[pl.ds(cid(t) * rb, rb)], semo[s])
                wait_out(s)
                fetch(cid(t + depth), s)
            return 0

        lax.fori_loop(0, ntd, body, 0, unroll=False)
        for s in range(depth):
            wait_gather(s)

    return rg


def _rgather(table, idx):
    v, d = table.shape
    bp = idx.shape[0]
    pick = None
    for depth in (4, 2):
        for rb in (512, 448, 256, 224, 128, 112, 64, 56):
            if depth * 4 * (rb * d + rb) <= _SC_BUDGET and bp % rb == 0:
                if pick is None or rb >= 128:
                    pick = (rb, depth)
                break
        if pick and pick[0] >= 128:
            break
    rb, depth = pick
    return _rgather_fn(v, d, bp, rb, depth)(table, idx)


@functools.cache
def _mpgather_fn(ns, ct, np_, r, depth):
    nchunks = np_ // r
    nt = -(-nchunks // NW)
    cblocks = ct // 16
    mesh = plsc.VectorSubcoreMesh(core_axis_name="c", subcore_axis_name="s")

    @functools.partial(
        pl.kernel,
        out_type=jax.ShapeDtypeStruct((np_, ct), jnp.float32),
        mesh=mesh,
        scratch_types=[
            [pltpu.VMEM((r * K,), jnp.int32)] * depth,
            [pltpu.VMEM((r * K, ct), jnp.float32)] * depth,
            [pltpu.VMEM((r, ct), jnp.float32)] * depth,
            [pltpu.SemaphoreType.DMA] * depth,
            [pltpu.SemaphoreType.DMA] * depth,
        ],
        compiler_params=_SC_PARAMS,
    )
    def mp(table_hbm, idx_hbm, out_hbm, idx_v, rows_v, obuf, semg, semo):
        wid = lax.axis_index("s") * 2 + lax.axis_index("c")

        def fetch(c, s):
            pltpu.sync_copy(idx_hbm.at[pl.ds(c * (r * K), r * K)], idx_v[s])
            pltpu.async_copy(table_hbm.at[idx_v[s]], rows_v[s], semg[s])

        def wait_gather(s):
            pltpu.make_async_copy(
                table_hbm.at[idx_v[s]], rows_v[s], semg[s]).wait()

        def issue_out(c, s):
            pltpu.async_copy(obuf[s], out_hbm.at[pl.ds(c * r, r)], semo[s])

        def wait_out(s):
            pltpu.make_async_copy(
                obuf[s], out_hbm.at[pl.ds(0, r)], semo[s]).wait()

        def work(s):
            rv, ob = rows_v[s], obuf[s]

            def j_body(j, _):
                def c_body(cb, _):
                    m = rv[j * K, pl.ds(cb * 16, 16)]
                    for k in range(1, K):
                        m = jnp.maximum(m, rv[j * K + k, pl.ds(cb * 16, 16)])
                    ob[j, pl.ds(cb * 16, 16)] = m
                    return 0

                lax.fori_loop(0, cblocks, c_body, 0, unroll=False)
                return 0

            lax.fori_loop(0, r, j_body, 0, unroll=False)

        _run_pipeline(nt, nchunks, depth, wid, fetch, wait_gather, wait_out,
                      work, issue_out)

    return mp


def _mpgather(table, idx_flat):
    ns, ct = table.shape
    np_ = idx_flat.shape[0] // K
    pick = None
    for depth in (4, 2):
        for r in (64, 32, 16, 8):
            if depth * 4 * (r * K * ct + r * ct + r * K) <= _SC_BUDGET \
                    and np_ % r == 0:
                pick = (r, depth)
                break
        if pick:
            break
    r, depth = pick
    return _mpgather_fn(ns, ct, np_, r, depth)(table, idx_flat)


def _kpconv_call(mode, gt, qt, kptt, wflat, bc, sigma, c, o,
                 extras, np_, bn, cout):
    cu = gt.shape[1]
    if mode == "b0":
        poff, feat_off = 0, 3
    else:
        poff = c

    def body(*refs):
        if mode == "b0":
            gt_ref, q_ref, kptt_ref, wflat_ref, bc_ref, o_ref = refs
        elif mode == "ws":
            (gt_ref, q_ref, kptt_ref, wflat_ref, bc_ref,
             x_ref, w2_ref, b2_ref, ws_ref, bs_ref, o_ref) = refs
        else:
            (gt_ref, q_ref, kptt_ref, wflat_ref, bc_ref,
             w2_ref, b2_ref, sc_ref, o_ref) = refs

        gt_b = gt_ref[...]
        qt_b = q_ref[...]
        kptt_b = kptt_ref[...]
        pt = gt_b[:, poff:poff + 3, :]
        rel = pt - qt_b[None, :, :]
        sq = None
        for d in range(3):
            reld = jnp.broadcast_to(rel[:, d:d + 1, :],
                                    (K, KP, bn)).reshape(K * KP, bn)
            diff = reld - kptt_b[:, d:d + 1]
            sq = diff * diff if sq is None else sq + diff * diff
        infl = jnp.maximum(0.0, 1.0 - jnp.sqrt(sq + 1e-12) / sigma)

        if mode == "b0":
            yg = gt_b[:, feat_off:feat_off + 1, :]
        else:
            yg = gt_b[:, 0:c, :]
        rows = []
        for p in range(KP):
            acc = infl[p, :][None, :] * yg[0]
            for k in range(1, K):
                acc = acc + infl[k * KP + p, :][None, :] * yg[k]
            rows.append(acc)
        wt = jnp.concatenate(rows, axis=0)
        y = lax.dot_general(wt, wflat_ref[...],
                            (((0,), (0,)), ((), ())),
                            preferred_element_type=jnp.float32,
                            precision=lax.Precision.HIGHEST)
        y = _lrelu(y + bc_ref[...])
        if mode == "b0":
            o_ref[...] = y
            return
        y = jnp.dot(y, w2_ref[...],
                    preferred_element_type=jnp.float32,
                    precision=lax.Precision.HIGHEST) + b2_ref[...]
        if mode == "ws":
            scp = jnp.dot(x_ref[...], ws_ref[...],
                          preferred_element_type=jnp.float32,
                          precision=lax.Precision.HIGHEST) + bs_ref[...]
        else:
            scp = sc_ref[...]
        o_ref[...] = _lrelu(y + scp)

    cc = 1 if mode == "b0" else c
    in_specs = [
        pl.BlockSpec((K, cu, bn), lambda i: (0, 0, i)),
        pl.BlockSpec((3, bn), lambda i: (0, i)),
        pl.BlockSpec((K * KP, 3), lambda i: (0, 0)),
        pl.BlockSpec((KP * cc, o), lambda i: (0, 0)),
        pl.BlockSpec((1, o), lambda i: (0, 0)),
    ]
    args = [gt, qt, kptt, wflat, bc]
    if mode == "ws":
        x, w2, b2, ws, bs = extras
        cin = x.shape[1]
        in_specs += [
            pl.BlockSpec((bn, cin), lambda i: (i, 0)),
            pl.BlockSpec((o, cout), lambda i: (0, 0)),
            pl.BlockSpec((1, cout), lambda i: (0, 0)),
            pl.BlockSpec((cin, cout), lambda i: (0, 0)),
            pl.BlockSpec((1, cout), lambda i: (0, 0)),
        ]
        args += [x, w2, b2.reshape(1, -1), ws, bs.reshape(1, -1)]
    elif mode == "pool":
        w2, b2, scrow = extras
        in_specs += [
            pl.BlockSpec((o, cout), lambda i: (0, 0)),
            pl.BlockSpec((1, cout), lambda i: (0, 0)),
            pl.BlockSpec((bn, cout), lambda i: (i, 0)),
        ]
        args += [w2, b2.reshape(1, -1), scrow]

    return pl.pallas_call(
        body,
        grid=(np_ // bn,),
        in_specs=in_specs,
        out_specs=pl.BlockSpec((bn, cout), lambda i: (i, 0)),
        out_shape=jax.ShapeDtypeStruct((np_, cout), jnp.float32),
    )(*args)


def _unary_call(xs, ws, b, act, bn):
    np_ = xs[0].shape[0]
    o = ws[0].shape[1]

    def body(*refs):
        n_in = len(xs)
        acc = refs[2 * n_in][...]
        for i in range(n_in):
            acc = acc + jnp.dot(refs[i][...], refs[n_in + i][...],
                                preferred_element_type=jnp.float32,
                                precision=lax.Precision.HIGHEST)
        refs[-1][...] = _lrelu(acc) if act else acc

    in_specs = [pl.BlockSpec((bn, x.shape[1]), lambda i: (i, 0)) for x in xs]
    in_specs += [pl.BlockSpec(w.shape, lambda i: (0, 0)) for w in ws]
    in_specs += [pl.BlockSpec((1, o), lambda i: (0, 0))]
    return pl.pallas_call(
        body,
        grid=(np_ // bn,),
        in_specs=in_specs,
        out_specs=pl.BlockSpec((bn, o), lambda i: (i, 0)),
        out_shape=jax.ShapeDtypeStruct((np_, o), jnp.float32),
    )(*xs, *ws, b.reshape(1, -1))


def _pad_rows(a, n):
    return jnp.pad(a, ((0, n - a.shape[0]),) + ((0, 0),) * (a.ndim - 1))


def _mktable(parts, ctot):
    t = jnp.concatenate(parts, axis=1)
    return jnp.pad(t, ((0, 0), (0, ctot - t.shape[1])))


def kernel(features, points0, points1, points2, params, neighbors0,
           neighbors1, neighbors2, pools0, pools1, upsamples0, upsamples1):
    p = params
    kp0 = p['kp']

    nb0 = _pad_rows(neighbors0, N0P).reshape(-1)
    nb1 = _pad_rows(neighbors1, N1P).reshape(-1)
    nb2 = _pad_rows(neighbors2, N2P).reshape(-1)
    pl0 = _pad_rows(pools0, N1P).reshape(-1)
    pl1 = _pad_rows(pools1, N2P).reshape(-1)
    up0 = _pad_rows(upsamples0[:, 0], N0P)
    up1 = _pad_rows(upsamples1[:, 0], N1P)

    pts0 = _pad_rows(points0, N0P)
    pts1 = _pad_rows(points1, N1P)
    pts2 = _pad_rows(points2, N2P)
    feat = _pad_rows(features, N0P)

    consts = []
    for lvl in range(3):
        kpts = kp0 * (2.0 ** lvl)
        sig = 0.3 * (2.0 ** lvl)
        kptt = jnp.tile(kpts, (K, 1))
        consts.append((kptt, sig))

    def wflat(wc):
        return wc.reshape(KP * wc.shape[1], wc.shape[2])

    bn0, bn1, bn2 = 896, 896, 640

    t0 = _mktable([pts0, feat], 16)
    qt0 = jnp.transpose(pts0)
    qt1 = jnp.transpose(pts1)
    qt2 = jnp.transpose(pts2)
    gt_b0 = _tgather(t0, nb0, 4, K)
    kptt0, s0 = consts[0]
    x0 = _kpconv_call("b0", gt_b0, qt0, kptt0,
                      p['b0']['w'].reshape(KP, 32), p['b0']['b'].reshape(1, -1),
                      s0, 1, 32, None, N0P, bn0, 32)

    y1 = _unary_call([x0], [p['b1']['w1']], p['b1']['b1'], True, bn0)
    gt1 = _tgather(_mktable([y1, pts0], 32), nb0, 19, K)
    x1 = _kpconv_call("ws", gt1, qt0, kptt0, wflat(p['b1']['wc']),
                      p['b1']['bc'].reshape(1, -1), s0, 16, 16,
                      (x0, p['b1']['w2'], p['b1']['b2'], p['b1']['ws'],
                       p['b1']['bs']), N0P, bn0, 64)

    y2 = _unary_call([x1], [p['b2']['w1']], p['b2']['b1'], True, bn0)
    gt2 = _tgather(_mktable([y2, pts0], 32), pl0, 19, K)
    sc2 = _mpgather(x1, pl0)
    x2 = _kpconv_call("pool", gt2, qt1, kptt0, wflat(p['b2']['wc']),
                      p['b2']['bc'].reshape(1, -1), s0, 16, 16,
                      (p['b2']['w2'], p['b2']['b2'], sc2), N1P, bn1, 64)

    y3 = _unary_call([x2], [p['b3']['w1']], p['b3']['b1'], True, bn1)
    gt3 = _tgather(_mktable([y3, pts1], 48), nb1, 35, K)
    kptt1, s1 = consts[1]
    x3 = _kpconv_call("ws", gt3, qt1, kptt1, wflat(p['b3']['wc']),
                      p['b3']['bc'].reshape(1, -1), s1, 32, 32,
                      (x2, p['b3']['w2'], p['b3']['b2'], p['b3']['ws'],
                       p['b3']['bs']), N1P, bn1, 128)

    y4 = _unary_call([x3], [p['b4']['w1']], p['b4']['b1'], True, bn1)
    gt4 = _tgather(_mktable([y4, pts1], 48), pl1, 35, K)
    sc4 = _mpgather(x3, pl1)
    x4 = _kpconv_call("pool", gt4, qt2, kptt1, wflat(p['b4']['wc']),
                      p['b4']['bc'].reshape(1, -1), s1, 32, 32,
                      (p['b4']['w2'], p['b4']['b2'], sc4), N2P, bn2, 128)

    y5 = _unary_call([x4], [p['b5']['w1']], p['b5']['b1'], True, bn2)
    gt5 = _tgather(_mktable([y5, pts2], 80), nb2, 67, K)
    kptt2, s2 = consts[2]
    x5 = _kpconv_call("ws", gt5, qt2, kptt2, wflat(p['b5']['wc']),
                      p['b5']['bc'].reshape(1, -1), s2, 64, 64,
                      (x4, p['b5']['w2'], p['b5']['b2'], p['b5']['ws'],
                       p['b5']['bs']), N2P, bn2, 256)

    u1 = _rgather(x5, up1)
    d1 = _unary_call([u1, x3], [p['d1']['w'][:256], p['d1']['w'][256:]],
                     p['d1']['b'], True, bn1)
    u0 = _rgather(d1, up0)
    out = _unary_call([u0, x1], [p['d3']['w'][:128], p['d3']['w'][128:]],
                      p['d3']['b'], False, bn0)
    return out[:N0]


# speedup vs baseline: 1.1959x; 1.1662x over previous
import functools
import math

import jax
import jax.numpy as jnp
from jax import lax
from jax.experimental import pallas as pl
from jax.experimental.pallas import tpu as pltpu
from jax.experimental.pallas import tpu_sc as plsc

K = 16
KP = 15
N0, N1, N2 = 50000, 12500, 3125
N0P, N1P, N2P = 50176, 12544, 3200
NW = 32


def _lrelu(x):
    return jnp.where(x >= 0, x, 0.1 * x)


@functools.cache
def _tgather_fn(ns, ct, cu, kk, np_, r):
    nchunks = np_ // r
    cpw = -(-nchunks // NW)
    jblocks = r // 16

    mesh = plsc.VectorSubcoreMesh(core_axis_name="c", subcore_axis_name="s")

    @functools.partial(
        pl.kernel,
        out_type=jax.ShapeDtypeStruct((kk, cu, np_), jnp.float32),
        mesh=mesh,
        scratch_types=[
            pltpu.VMEM((r * kk,), jnp.int32),
            pltpu.VMEM((r * kk, ct), jnp.float32),
            pltpu.VMEM((kk, cu, r), jnp.float32),
            pltpu.SemaphoreType.DMA,
        ],
        compiler_params=pltpu.CompilerParams(
            use_tc_tiling_on_sc=False, needs_layout_passes=False),
    )
    def tg(table_hbm, idx_hbm, out_hbm, idx_v, rows_v, obuf, sem):
        wid = lax.axis_index("s") * 2 + lax.axis_index("c")
        lane = lax.iota(jnp.int32, 16)

        def chunk_body(t, _):
            cid = wid * cpw + t

            @pl.when(cid < nchunks)
            def _():
                n0 = cid * r
                pltpu.sync_copy(idx_hbm.at[pl.ds(n0 * kk, r * kk)], idx_v)
                pltpu.async_copy(table_hbm.at[idx_v], rows_v, sem).wait()

                def c_body(c, _):
                    cvec = jnp.full((16,), 0, jnp.int32) + c

                    def j_body(jb, _):
                        rbase = lane * kk + jb * (16 * kk)
                        for k in range(kk):
                            v = plsc.load_gather(rows_v, [rbase + k, cvec])
                            obuf[k, c, pl.ds(jb * 16, 16)] = v
                        return 0

                    lax.fori_loop(0, jblocks, j_body, 0, unroll=False)
                    return 0

                lax.fori_loop(0, cu, c_body, 0, unroll=False)
                pltpu.sync_copy(obuf, out_hbm.at[:, :, pl.ds(n0, r)])

            return 0

        lax.fori_loop(0, cpw, chunk_body, 0, unroll=False)

    return tg


def _tgather(table, idx_flat, cu, kk):
    ns, ct = table.shape
    np_ = idx_flat.shape[0] // kk
    budget = 384 * 1024
    r = 16
    for cand in (128, 64, 32, 16):
        if (cand * kk * ct + kk * cu * cand) * 4 <= budget and np_ % cand == 0:
            r = cand
            break
    return _tgather_fn(ns, ct, cu, kk, np_, r)(table, idx_flat)


@functools.cache
def _rgather_fn(v, d, bp, rb):
    nchunks = bp // rb
    cpw = -(-nchunks // NW)
    mesh = plsc.VectorSubcoreMesh(core_axis_name="c", subcore_axis_name="s")

    @functools.partial(
        pl.kernel,
        out_type=jax.ShapeDtypeStruct((bp, d), jnp.float32),
        mesh=mesh,
        scratch_types=[
            pltpu.VMEM((rb,), jnp.int32),
            pltpu.VMEM((rb, d), jnp.float32),
            pltpu.SemaphoreType.DMA,
        ],
        compiler_params=pltpu.CompilerParams(
            use_tc_tiling_on_sc=False, needs_layout_passes=False),
    )
    def rg(table_hbm, idx_hbm, out_hbm, idx_v, rows_v, sem):
        wid = lax.axis_index("s") * 2 + lax.axis_index("c")

        def chunk_body(t, _):
            cid = wid * cpw + t

            @pl.when(cid < nchunks)
            def _():
                n0 = cid * rb
                pltpu.sync_copy(idx_hbm.at[pl.ds(n0, rb)], idx_v)
                pltpu.async_copy(table_hbm.at[idx_v], rows_v, sem).wait()
                pltpu.sync_copy(rows_v, out_hbm.at[pl.ds(n0, rb)])

            return 0

        lax.fori_loop(0, cpw, chunk_body, 0, unroll=False)

    return rg


def _rgather(table, idx):
    v, d = table.shape
    bp = idx.shape[0]
    rb = 256 if d > 128 else 512
    while bp % rb:
        rb //= 2
    return _rgather_fn(v, d, bp, rb)(table, idx)


def _kpconv_call(mode, gt, qt, kptt, wflat, bc, sigma, c, o,
                 extras, np_, bn, cout):
    cu = gt.shape[1]
    if mode == "b0":
        poff, feat_off = 0, 3
    elif mode == "ws":
        poff = c
    else:
        cs = extras[2].shape[0]
        poff = c + cs

    def body(*refs):
        if mode == "b0":
            gt_ref, qt_ref, kptt_ref, wflat_ref, bc_ref, o_ref = refs
        elif mode == "ws":
            (gt_ref, qt_ref, kptt_ref, wflat_ref, bc_ref,
             x_ref, w2_ref, b2_ref, ws_ref, bs_ref, o_ref) = refs
        else:
            (gt_ref, qt_ref, kptt_ref, wflat_ref, bc_ref,
             w2_ref, b2_ref, eye_ref, o_ref) = refs

        gt_b = gt_ref[...]
        qt_b = qt_ref[...]
        kptt_b = kptt_ref[...]
        pt = gt_b[:, poff:poff + 3, :]
        rel = pt - qt_b[None, :, :]
        sq = None
        for d in range(3):
            reld = jnp.broadcast_to(rel[:, d:d + 1, :],
                                    (K, KP, bn)).reshape(K * KP, bn)
            diff = reld - kptt_b[:, d:d + 1]
            sq = diff * diff if sq is None else sq + diff * diff
        infl = jnp.maximum(0.0, 1.0 - jnp.sqrt(sq + 1e-12) / sigma)

        if mode == "b0":
            yg = gt_b[:, feat_off:feat_off + 1, :]
        else:
            yg = gt_b[:, 0:c, :]
        rows = []
        for p in range(KP):
            acc = infl[p, :][None, :] * yg[0]
            for k in range(1, K):
                acc = acc + infl[k * KP + p, :][None, :] * yg[k]
            rows.append(acc)
        wt = jnp.concatenate(rows, axis=0)
        y = lax.dot_general(wt, wflat_ref[...],
                            (((0,), (0,)), ((), ())),
                            preferred_element_type=jnp.float32,
                    precision=lax.Precision.HIGHEST)
        y = _lrelu(y + bc_ref[...])
        if mode == "b0":
            o_ref[...] = y
            return
        y = jnp.dot(y, w2_ref[...],
                    preferred_element_type=jnp.float32,
                    precision=lax.Precision.HIGHEST) + b2_ref[...]
        if mode == "ws":
            scp = jnp.dot(x_ref[...], ws_ref[...],
                          preferred_element_type=jnp.float32,
                    precision=lax.Precision.HIGHEST) + bs_ref[...]
        else:
            xg = gt_b[:, c:c + cs, :]
            sct = jnp.max(xg, axis=0)
            scp = lax.dot_general(sct, eye_ref[...],
                                  (((0,), (0,)), ((), ())),
                                  preferred_element_type=jnp.float32,
                    precision=lax.Precision.HIGHEST)
        o_ref[...] = _lrelu(y + scp)

    cc = 1 if mode == "b0" else c
    in_specs = [
        pl.BlockSpec((K, cu, bn), lambda i: (0, 0, i)),
        pl.BlockSpec((3, bn), lambda i: (0, i)),
        pl.BlockSpec((K * KP, 3), lambda i: (0, 0)),
        pl.BlockSpec((KP * cc, o), lambda i: (0, 0)),
        pl.BlockSpec((1, o), lambda i: (0, 0)),
    ]
    args = [gt, qt, kptt, wflat, bc]
    if mode == "ws":
        x, w2, b2, ws, bs = extras
        cin = x.shape[1]
        in_specs += [
            pl.BlockSpec((bn, cin), lambda i: (i, 0)),
            pl.BlockSpec((o, cout), lambda i: (0, 0)),
            pl.BlockSpec((1, cout), lambda i: (0, 0)),
            pl.BlockSpec((cin, cout), lambda i: (0, 0)),
            pl.BlockSpec((1, cout), lambda i: (0, 0)),
        ]
        args += [x, w2, b2.reshape(1, -1), ws, bs.reshape(1, -1)]
    elif mode == "pool":
        w2, b2, eye_cs = extras
        in_specs += [
            pl.BlockSpec((o, cout), lambda i: (0, 0)),
            pl.BlockSpec((1, cout), lambda i: (0, 0)),
            pl.BlockSpec((cs, cs), lambda i: (0, 0)),
        ]
        args += [w2, b2.reshape(1, -1), eye_cs]

    return pl.pallas_call(
        body,
        grid=(np_ // bn,),
        in_specs=in_specs,
        out_specs=pl.BlockSpec((bn, cout), lambda i: (i, 0)),
        out_shape=jax.ShapeDtypeStruct((np_, cout), jnp.float32),
    )(*args)


def _unary_call(xs, ws, b, act, bn):
    np_ = xs[0].shape[0]
    o = ws[0].shape[1]

    def body(*refs):
        n_in = len(xs)
        acc = refs[2 * n_in][...]
        for i in range(n_in):
            acc = acc + jnp.dot(refs[i][...], refs[n_in + i][...],
                                preferred_element_type=jnp.float32,
                    precision=lax.Precision.HIGHEST)
        refs[-1][...] = _lrelu(acc) if act else acc

    in_specs = [pl.BlockSpec((bn, x.shape[1]), lambda i: (i, 0)) for x in xs]
    in_specs += [pl.BlockSpec(w.shape, lambda i: (0, 0)) for w in ws]
    in_specs += [pl.BlockSpec((1, o), lambda i: (0, 0))]
    return pl.pallas_call(
        body,
        grid=(np_ // bn,),
        in_specs=in_specs,
        out_specs=pl.BlockSpec((bn, o), lambda i: (i, 0)),
        out_shape=jax.ShapeDtypeStruct((np_, o), jnp.float32),
    )(*xs, *ws, b.reshape(1, -1))


def _pad_rows(a, n):
    return jnp.pad(a, ((0, n - a.shape[0]),) + ((0, 0),) * (a.ndim - 1))


def _mktable(parts, ctot):
    t = jnp.concatenate(parts, axis=1)
    return jnp.pad(t, ((0, 0), (0, ctot - t.shape[1])))


def kernel(features, points0, points1, points2, params, neighbors0,
           neighbors1, neighbors2, pools0, pools1, upsamples0, upsamples1):
    p = params
    kp0 = p['kp']

    nb0 = _pad_rows(neighbors0, N0P).reshape(-1)
    nb1 = _pad_rows(neighbors1, N1P).reshape(-1)
    nb2 = _pad_rows(neighbors2, N2P).reshape(-1)
    pl0 = _pad_rows(pools0, N1P).reshape(-1)
    pl1 = _pad_rows(pools1, N2P).reshape(-1)
    up0 = _pad_rows(upsamples0[:, 0], N0P)
    up1 = _pad_rows(upsamples1[:, 0], N1P)
    ia0 = jnp.arange(N0P, dtype=jnp.int32)
    ia1 = jnp.arange(N1P, dtype=jnp.int32)
    ia2 = jnp.arange(N2P, dtype=jnp.int32)

    pts0 = _pad_rows(points0, N0P)
    pts1 = _pad_rows(points1, N1P)
    pts2 = _pad_rows(points2, N2P)
    feat = _pad_rows(features, N0P)

    consts = []
    for lvl in range(3):
        kpts = kp0 * (2.0 ** lvl)
        sig = 0.3 * (2.0 ** lvl)
        kptt = jnp.tile(kpts, (K, 1))
        consts.append((kptt, sig))

    def wflat(wc):
        return wc.reshape(KP * wc.shape[1], wc.shape[2])

    bn0, bn1, bn2 = 896, 896, 640

    t0 = _mktable([pts0, feat], 16)
    qt0 = _tgather(t0, ia0, 3, 1).reshape(3, N0P)
    gt_b0 = _tgather(t0, nb0, 4, K)
    kptt0, s0 = consts[0]
    x0 = _kpconv_call("b0", gt_b0, qt0, kptt0,
                      p['b0']['w'].reshape(KP, 32), p['b0']['b'].reshape(1, -1),
                      s0, 1, 32, None, N0P, bn0, 32)

    y1 = _unary_call([x0], [p['b1']['w1']], p['b1']['b1'], True, bn0)
    gt1 = _tgather(_mktable([y1, pts0], 32), nb0, 19, K)
    x1 = _kpconv_call("ws", gt1, qt0, kptt0, wflat(p['b1']['wc']),
                      p['b1']['bc'].reshape(1, -1), s0, 16, 16,
                      (x0, p['b1']['w2'], p['b1']['b2'], p['b1']['ws'],
                       p['b1']['bs']), N0P, bn0, 64)

    y2 = _unary_call([x1], [p['b2']['w1']], p['b2']['b1'], True, bn0)
    qt1 = _tgather(_mktable([pts1], 16), ia1, 3, 1).reshape(3, N1P)
    gt2 = _tgather(_mktable([y2, x1, pts0], 96), pl0, 83, K)
    x2 = _kpconv_call("pool", gt2, qt1, kptt0, wflat(p['b2']['wc']),
                      p['b2']['bc'].reshape(1, -1), s0, 16, 16,
                      (p['b2']['w2'], p['b2']['b2'],
                       jnp.eye(64, dtype=jnp.float32)), N1P, bn1, 64)

    y3 = _unary_call([x2], [p['b3']['w1']], p['b3']['b1'], True, bn1)
    gt3 = _tgather(_mktable([y3, pts1], 48), nb1, 35, K)
    kptt1, s1 = consts[1]
    x3 = _kpconv_call("ws", gt3, qt1, kptt1, wflat(p['b3']['wc']),
                      p['b3']['bc'].reshape(1, -1), s1, 32, 32,
                      (x2, p['b3']['w2'], p['b3']['b2'], p['b3']['ws'],
                       p['b3']['bs']), N1P, bn1, 128)

    y4 = _unary_call([x3], [p['b4']['w1']], p['b4']['b1'], True, bn1)
    qt2 = _tgather(_mktable([pts2], 16), ia2, 3, 1).reshape(3, N2P)
    gt4 = _tgather(_mktable([y4, x3, pts1], 176), pl1, 163, K)
    x4 = _kpconv_call("pool", gt4, qt2, kptt1, wflat(p['b4']['wc']),
                      p['b4']['bc'].reshape(1, -1), s1, 32, 32,
                      (p['b4']['w2'], p['b4']['b2'],
                       jnp.eye(128, dtype=jnp.float32)), N2P, bn2, 128)

    y5 = _unary_call([x4], [p['b5']['w1']], p['b5']['b1'], True, bn2)
    gt5 = _tgather(_mktable([y5, pts2], 80), nb2, 67, K)
    kptt2, s2 = consts[2]
    x5 = _kpconv_call("ws", gt5, qt2, kptt2, wflat(p['b5']['wc']),
                      p['b5']['bc'].reshape(1, -1), s2, 64, 64,
                      (x4, p['b5']['w2'], p['b5']['b2'], p['b5']['ws'],
                       p['b5']['bs']), N2P, bn2, 256)

    u1 = _rgather(x5, up1)
    d1 = _unary_call([u1, x3], [p['d1']['w'][:256], p['d1']['w'][256:]],
                     p['d1']['b'], True, bn1)
    u0 = _rgather(d1, up0)
    out = _unary_call([u0, x1], [p['d3']['w'][:128], p['d3']['w'][128:]],
                      p['d3']['b'], False, bn0)
    return out[:N0]


# speedup vs baseline: 1.3180x; 1.1021x over previous
import functools

import jax
import jax.numpy as jnp
from jax import lax
from jax.experimental import pallas as pl
from jax.experimental.pallas import tpu as pltpu
from jax.experimental.pallas import tpu_sc as plsc

K = 16
KP = 15
N0, N1, N2 = 50000, 12500, 3125
N0P, N1P, N2P = 50176, 12544, 3200
NW = 32


def _lrelu(x):
    return jnp.where(x >= 0, x, 0.1 * x)


@functools.cache
def _tgather_fn(ns, ct, cu, kk, np_, r):
    nchunks = np_ // r
    cpw = -(-nchunks // NW)
    jblocks = r // 16

    mesh = plsc.VectorSubcoreMesh(core_axis_name="c", subcore_axis_name="s")

    @functools.partial(
        pl.kernel,
        out_type=jax.ShapeDtypeStruct((kk, cu, np_), jnp.float32),
        mesh=mesh,
        scratch_types=[
            pltpu.VMEM((r * kk,), jnp.int32),
            pltpu.VMEM((r * kk, ct), jnp.float32),
            pltpu.VMEM((kk, cu, r), jnp.float32),
            pltpu.SemaphoreType.DMA,
        ],
        compiler_params=pltpu.CompilerParams(
            use_tc_tiling_on_sc=False, needs_layout_passes=False),
    )
    def tg(table_hbm, idx_hbm, out_hbm, idx_v, rows_v, obuf, sem):
        wid = lax.axis_index("s") * 2 + lax.axis_index("c")
        lane = lax.iota(jnp.int32, 16)

        def chunk_body(t, _):
            cid = wid * cpw + t

            @pl.when(cid < nchunks)
            def _():
                n0 = cid * r
                pltpu.sync_copy(idx_hbm.at[pl.ds(n0 * kk, r * kk)], idx_v)
                pltpu.async_copy(table_hbm.at[idx_v], rows_v, sem).wait()

                def c_body(c, _):
                    cvec = jnp.full((16,), 0, jnp.int32) + c

                    def j_body(jb, _):
                        rbase = lane * kk + jb * (16 * kk)
                        for k in range(kk):
                            v = plsc.load_gather(rows_v, [rbase + k, cvec])
                            obuf[k, c, pl.ds(jb * 16, 16)] = v
                        return 0

                    lax.fori_loop(0, jblocks, j_body, 0, unroll=False)
                    return 0

                lax.fori_loop(0, cu, c_body, 0, unroll=False)
                pltpu.sync_copy(obuf, out_hbm.at[:, :, pl.ds(n0, r)])

            return 0

        lax.fori_loop(0, cpw, chunk_body, 0, unroll=False)

    return tg


def _tgather(table, idx_flat, cu, kk):
    ns, ct = table.shape
    np_ = idx_flat.shape[0] // kk
    budget = 384 * 1024
    r = 16
    for cand in (128, 64, 32, 16):
        if (cand * kk * ct + kk * cu * cand) * 4 <= budget and np_ % cand == 0:
            r = cand
            break
    return _tgather_fn(ns, ct, cu, kk, np_, r)(table, idx_flat)


@functools.cache
def _rgather_fn(v, d, bp, rb):
    nchunks = bp // rb
    cpw = -(-nchunks // NW)
    mesh = plsc.VectorSubcoreMesh(core_axis_name="c", subcore_axis_name="s")

    @functools.partial(
        pl.kernel,
        out_type=jax.ShapeDtypeStruct((bp, d), jnp.float32),
        mesh=mesh,
        scratch_types=[
            pltpu.VMEM((rb,), jnp.int32),
            pltpu.VMEM((rb, d), jnp.float32),
            pltpu.SemaphoreType.DMA,
        ],
        compiler_params=pltpu.CompilerParams(
            use_tc_tiling_on_sc=False, needs_layout_passes=False),
    )
    def rg(table_hbm, idx_hbm, out_hbm, idx_v, rows_v, sem):
        wid = lax.axis_index("s") * 2 + lax.axis_index("c")

        def chunk_body(t, _):
            cid = wid * cpw + t

            @pl.when(cid < nchunks)
            def _():
                n0 = cid * rb
                pltpu.sync_copy(idx_hbm.at[pl.ds(n0, rb)], idx_v)
                pltpu.async_copy(table_hbm.at[idx_v], rows_v, sem).wait()
                pltpu.sync_copy(rows_v, out_hbm.at[pl.ds(n0, rb)])

            return 0

        lax.fori_loop(0, cpw, chunk_body, 0, unroll=False)

    return rg


def _rgather(table, idx):
    v, d = table.shape
    bp = idx.shape[0]
    rb = 256 if d > 128 else 512
    while bp % rb:
        rb //= 2
    return _rgather_fn(v, d, bp, rb)(table, idx)


def _kpconv_call(mode, gt, qt, kptt, wflat, bc, sigma, c, o,
                 extras, np_, bn, cout):
    cu = gt.shape[1]
    if mode == "b0":
        poff, feat_off = 0, 3
    elif mode == "ws":
        poff = c
    else:
        cs = extras[2].shape[0]
        poff = c + cs

    def body(*refs):
        if mode == "b0":
            gt_ref, qt_ref, kptt_ref, wflat_ref, bc_ref, o_ref = refs
        elif mode == "ws":
            (gt_ref, qt_ref, kptt_ref, wflat_ref, bc_ref,
             x_ref, w2_ref, b2_ref, ws_ref, bs_ref, o_ref) = refs
        else:
            (gt_ref, qt_ref, kptt_ref, wflat_ref, bc_ref,
             w2_ref, b2_ref, eye_ref, o_ref) = refs

        gt_b = gt_ref[...]
        qt_b = qt_ref[...]
        kptt_b = kptt_ref[...]
        pt = gt_b[:, poff:poff + 3, :]
        rel = pt - qt_b[None, :, :]
        sq = None
        for d in range(3):
            reld = jnp.broadcast_to(rel[:, d:d + 1, :],
                                    (K, KP, bn)).reshape(K * KP, bn)
            diff = reld - kptt_b[:, d:d + 1]
            sq = diff * diff if sq is None else sq + diff * diff
        infl = jnp.maximum(0.0, 1.0 - jnp.sqrt(sq + 1e-12) / sigma)

        if mode == "b0":
            yg = gt_b[:, feat_off:feat_off + 1, :]
        else:
            yg = gt_b[:, 0:c, :]
        rows = []
        for p in range(KP):
            acc = infl[p, :][None, :] * yg[0]
            for k in range(1, K):
                acc = acc + infl[k * KP + p, :][None, :] * yg[k]
            rows.append(acc)
        wt = jnp.concatenate(rows, axis=0)
        y = lax.dot_general(wt, wflat_ref[...],
                            (((0,), (0,)), ((), ())),
                            preferred_element_type=jnp.float32)
        y = _lrelu(y + bc_ref[...])
        if mode == "b0":
            o_ref[...] = y
            return
        y = jnp.dot(y, w2_ref[...],
                    preferred_element_type=jnp.float32) + b2_ref[...]
        if mode == "ws":
            scp = jnp.dot(x_ref[...], ws_ref[...],
                          preferred_element_type=jnp.float32) + bs_ref[...]
        else:
            xg = gt_b[:, c:c + cs, :]
            sct = jnp.max(xg, axis=0)
            scp = lax.dot_general(sct, eye_ref[...],
                                  (((0,), (0,)), ((), ())),
                                  preferred_element_type=jnp.float32)
        o_ref[...] = _lrelu(y + scp)

    cc = 1 if mode == "b0" else c
    in_specs = [
        pl.BlockSpec((K, cu, bn), lambda i: (0, 0, i)),
        pl.BlockSpec((3, bn), lambda i: (0, i)),
        pl.BlockSpec((K * KP, 3), lambda i: (0, 0)),
        pl.BlockSpec((KP * cc, o), lambda i: (0, 0)),
        pl.BlockSpec((1, o), lambda i: (0, 0)),
    ]
    args = [gt, qt, kptt, wflat, bc]
    if mode == "ws":
        x, w2, b2, ws, bs = extras
        cin = x.shape[1]
        in_specs += [
            pl.BlockSpec((bn, cin), lambda i: (i, 0)),
            pl.BlockSpec((o, cout), lambda i: (0, 0)),
            pl.BlockSpec((1, cout), lambda i: (0, 0)),
            pl.BlockSpec((cin, cout), lambda i: (0, 0)),
            pl.BlockSpec((1, cout), lambda i: (0, 0)),
        ]
        args += [x, w2, b2.reshape(1, -1), ws, bs.reshape(1, -1)]
    elif mode == "pool":
        w2, b2, eye_cs = extras
        in_specs += [
            pl.BlockSpec((o, cout), lambda i: (0, 0)),
            pl.BlockSpec((1, cout), lambda i: (0, 0)),
            pl.BlockSpec((cs, cs), lambda i: (0, 0)),
        ]
        args += [w2, b2.reshape(1, -1), eye_cs]

    return pl.pallas_call(
        body,
        grid=(np_ // bn,),
        in_specs=in_specs,
        out_specs=pl.BlockSpec((bn, cout), lambda i: (i, 0)),
        out_shape=jax.ShapeDtypeStruct((np_, cout), jnp.float32),
    )(*args)


def _unary_call(xs, ws, b, act, bn):
    np_ = xs[0].shape[0]
    o = ws[0].shape[1]

    def body(*refs):
        n_in = len(xs)
        acc = refs[2 * n_in][...]
        for i in range(n_in):
            acc = acc + jnp.dot(refs[i][...], refs[n_in + i][...],
                                preferred_element_type=jnp.float32)
        refs[-1][...] = _lrelu(acc) if act else acc

    in_specs = [pl.BlockSpec((bn, x.shape[1]), lambda i: (i, 0)) for x in xs]
    in_specs += [pl.BlockSpec(w.shape, lambda i: (0, 0)) for w in ws]
    in_specs += [pl.BlockSpec((1, o), lambda i: (0, 0))]
    return pl.pallas_call(
        body,
        grid=(np_ // bn,),
        in_specs=in_specs,
        out_specs=pl.BlockSpec((bn, o), lambda i: (i, 0)),
        out_shape=jax.ShapeDtypeStruct((np_, o), jnp.float32),
    )(*xs, *ws, b.reshape(1, -1))


def _pad_rows(a, n):
    return jnp.pad(a, ((0, n - a.shape[0]),) + ((0, 0),) * (a.ndim - 1))


def _mktable(parts, ctot):
    t = jnp.concatenate(parts, axis=1)
    return jnp.pad(t, ((0, 0), (0, ctot - t.shape[1])))


def kernel(features, points0, points1, points2, params, neighbors0,
           neighbors1, neighbors2, pools0, pools1, upsamples0, upsamples1):
    p = params
    kp0 = p['kp']

    nb0 = _pad_rows(neighbors0, N0P).reshape(-1)
    nb1 = _pad_rows(neighbors1, N1P).reshape(-1)
    nb2 = _pad_rows(neighbors2, N2P).reshape(-1)
    pl0 = _pad_rows(pools0, N1P).reshape(-1)
    pl1 = _pad_rows(pools1, N2P).reshape(-1)
    up0 = _pad_rows(upsamples0[:, 0], N0P)
    up1 = _pad_rows(upsamples1[:, 0], N1P)
    ia0 = jnp.arange(N0P, dtype=jnp.int32)
    ia1 = jnp.arange(N1P, dtype=jnp.int32)
    ia2 = jnp.arange(N2P, dtype=jnp.int32)

    pts0 = _pad_rows(points0, N0P)
    pts1 = _pad_rows(points1, N1P)
    pts2 = _pad_rows(points2, N2P)
    feat = _pad_rows(features, N0P)

    consts = []
    for lvl in range(3):
        kpts = kp0 * (2.0 ** lvl)
        sig = 0.3 * (2.0 ** lvl)
        kptt = jnp.tile(kpts, (K, 1))
        consts.append((kptt, sig))

    def wflat(wc):
        return wc.reshape(KP * wc.shape[1], wc.shape[2])

    bn0, bn1, bn2 = 896, 896, 640

    t0 = _mktable([pts0, feat], 16)
    qt0 = _tgather(t0, ia0, 3, 1).reshape(3, N0P)
    gt_b0 = _tgather(t0, nb0, 4, K)
    kptt0, s0 = consts[0]
    x0 = _kpconv_call("b0", gt_b0, qt0, kptt0,
                      p['b0']['w'].reshape(KP, 32), p['b0']['b'].reshape(1, -1),
                      s0, 1, 32, None, N0P, bn0, 32)

    y1 = _unary_call([x0], [p['b1']['w1']], p['b1']['b1'], True, bn0)
    gt1 = _tgather(_mktable([y1, pts0], 32), nb0, 19, K)
    x1 = _kpconv_call("ws", gt1, qt0, kptt0, wflat(p['b1']['wc']),
                      p['b1']['bc'].reshape(1, -1), s0, 16, 16,
                      (x0, p['b1']['w2'], p['b1']['b2'], p['b1']['ws'],
                       p['b1']['bs']), N0P, bn0, 64)

    y2 = _unary_call([x1], [p['b2']['w1']], p['b2']['b1'], True, bn0)
    qt1 = _tgather(_mktable([pts1], 16), ia1, 3, 1).reshape(3, N1P)
    gt2 = _tgather(_mktable([y2, x1, pts0], 96), pl0, 83, K)
    x2 = _kpconv_call("pool", gt2, qt1, kptt0, wflat(p['b2']['wc']),
                      p['b2']['bc'].reshape(1, -1), s0, 16, 16,
                      (p['b2']['w2'], p['b2']['b2'],
                       jnp.eye(64, dtype=jnp.float32)), N1P, bn1, 64)

    y3 = _unary_call([x2], [p['b3']['w1']], p['b3']['b1'], True, bn1)
    gt3 = _tgather(_mktable([y3, pts1], 48), nb1, 35, K)
    kptt1, s1 = consts[1]
    x3 = _kpconv_call("ws", gt3, qt1, kptt1, wflat(p['b3']['wc']),
                      p['b3']['bc'].reshape(1, -1), s1, 32, 32,
                      (x2, p['b3']['w2'], p['b3']['b2'], p['b3']['ws'],
                       p['b3']['bs']), N1P, bn1, 128)

    y4 = _unary_call([x3], [p['b4']['w1']], p['b4']['b1'], True, bn1)
    qt2 = _tgather(_mktable([pts2], 16), ia2, 3, 1).reshape(3, N2P)
    gt4 = _tgather(_mktable([y4, x3, pts1], 176), pl1, 163, K)
    x4 = _kpconv_call("pool", gt4, qt2, kptt1, wflat(p['b4']['wc']),
                      p['b4']['bc'].reshape(1, -1), s1, 32, 32,
                      (p['b4']['w2'], p['b4']['b2'],
                       jnp.eye(128, dtype=jnp.float32)), N2P, bn2, 128)

    y5 = _unary_call([x4], [p['b5']['w1']], p['b5']['b1'], True, bn2)
    gt5 = _tgather(_mktable([y5, pts2], 80), nb2, 67, K)
    kptt2, s2 = consts[2]
    x5 = _kpconv_call("ws", gt5, qt2, kptt2, wflat(p['b5']['wc']),
                      p['b5']['bc'].reshape(1, -1), s2, 64, 64,
                      (x4, p['b5']['w2'], p['b5']['b2'], p['b5']['ws'],
                       p['b5']['bs']), N2P, bn2, 256)

    u1 = _rgather(x5, up1)
    d1 = _unary_call([u1, x3], [p['d1']['w'][:256], p['d1']['w'][256:]],
                     p['d1']['b'], True, bn1)
    u0 = _rgather(d1, up0)
    out = _unary_call([u0, x1], [p['d3']['w'][:128], p['d3']['w'][128:]],
                      p['d3']['b'], False, bn0)
    return out[:N0]
